# baseline (speedup 1.0000x reference)
import functools

import jax
import jax.numpy as jnp
from jax import lax
from jax.experimental import pallas as pl
from jax.experimental.pallas import tpu as pltpu
from jax.experimental.pallas import tpu_sc as plsc

N = 10000
E = 320000
F = 128
H = 256
NG = 64
AD = 32

NC = 2
NS = 16
NW = NC * NS
K = 128
NB = 80
NBH = NB // 2
EPWP = NB * K
PADW = EPWP - E // NW
NPAD = 10240
ZDEG = NPAD // NS
RPS = NPAD // NS
RB = 1000
NSTEP = N // RB

_mesh = plsc.VectorSubcoreMesh(core_axis_name="c", subcore_axis_name="s")


@functools.partial(
    pl.kernel,
    out_type=jax.ShapeDtypeStruct((NC, NPAD), jnp.float32),
    mesh=_mesh,
    scratch_types=[
        pltpu.VMEM((NB, K), jnp.int32),
        pltpu.VMEM((128,), jnp.float32),
        pltpu.VMEM((ZDEG,), jnp.float32),
        pltpu.VMEM_SHARED((NPAD,), jnp.float32),
    ],
)
def _deg_kernel(dst_hbm, out_hbm, idx_v, ones_v, zb_v, acc_sh):
    c = lax.axis_index("c")
    s = lax.axis_index("s")
    w = c * NS + s
    for i in range(8):
        ones_v[pl.ds(i * 16, 16)] = jnp.ones((16,), jnp.float32)
    for i in range(ZDEG // 16):
        zb_v[pl.ds(i * 16, 16)] = jnp.zeros((16,), jnp.float32)
    pltpu.sync_copy(dst_hbm.at[w], idx_v)
    pltpu.sync_copy(zb_v, acc_sh.at[pl.ds(s * ZDEG, ZDEG)])
    plsc.subcore_barrier()

    def body(j, carry):
        pltpu.sync_copy(ones_v, acc_sh.at[idx_v.at[j]], add=True)
        return carry

    lax.fori_loop(0, NB, body, 0)
    plsc.subcore_barrier()
    pltpu.sync_copy(acc_sh.at[pl.ds(s * ZDEG, ZDEG)],
                    out_hbm.at[c, pl.ds(s * ZDEG, ZDEG)])


def _make_spmv(ch_count):
    @functools.partial(
        pl.kernel,
        out_type=jax.ShapeDtypeStruct((ch_count, NC, NPAD, F), jnp.float32),
        mesh=_mesh,
        scratch_types=[
            pltpu.VMEM((NBH, K), jnp.int32),
            pltpu.VMEM((NBH, K), jnp.int32),
            pltpu.VMEM((2, K, F), jnp.float32),
            pltpu.VMEM_SHARED((NPAD, F), jnp.float32),
            pltpu.SemaphoreType.DMA,
            pltpu.SemaphoreType.DMA,
            pltpu.SemaphoreType.DMA,
            pltpu.SemaphoreType.DMA,
        ],
    )
    def _spmv(table_hbm, src_hbm, dst_hbm, zeros_hbm, out_hbm,
              src_v, dst_v, rows_v, acc_sh, sem0, sem1, ssem0, ssem1):
        c = lax.axis_index("c")
        s = lax.axis_index("s")
        w = c * NS + s
        sems = (sem0, sem1)

        for ch in range(ch_count):
            tab = table_hbm.at[ch]

            def gather(j, b):
                return pltpu.make_async_copy(tab.at[src_v.at[j]],
                                             rows_v.at[b], sems[b])

            pltpu.sync_copy(zeros_hbm.at[pl.ds(s * RPS, RPS)],
                            acc_sh.at[pl.ds(s * RPS, RPS)])
            plsc.subcore_barrier()

            for half in range(2):
                pltpu.sync_copy(src_hbm.at[w, pl.ds(half * NBH, NBH)], src_v)
                pltpu.sync_copy(dst_hbm.at[w, pl.ds(half * NBH, NBH)], dst_v)
                gather(0, 0).start()
                gather(1, 1).start()

                def body(i, carry):
                    j0 = 2 * i
                    gather(j0, 0).wait()
                    sc0 = pltpu.async_copy(rows_v.at[0],
                                           acc_sh.at[dst_v.at[j0]], ssem0,
                                           add=True)
                    gather(j0 + 1, 1).wait()
                    sc1 = pltpu.async_copy(rows_v.at[1],
                                           acc_sh.at[dst_v.at[j0 + 1]], ssem1,
                                           add=True)
                    sc0.wait()

                    @pl.when(j0 + 2 < NBH)
                    def _():
                        gather(j0 + 2, 0).start()

                    sc1.wait()

                    @pl.when(j0 + 3 < NBH)
                    def _():
                        gather(j0 + 3, 1).start()

                    return carry

                lax.fori_loop(0, NBH // 2, body, 0)
            plsc.subcore_barrier()
            pltpu.sync_copy(acc_sh.at[pl.ds(s * RPS, RPS)],
                            out_hbm.at[ch, c, pl.ds(s * RPS, RPS)])

    return _spmv


_spmv1 = _make_spmv(1)
_spmv4 = _make_spmv(4)


def _prescale_body(deg_ref, x_ref, xs_ref, dinv_ref):
    deg = deg_ref[0] + deg_ref[1] + 1.0
    dinv = lax.rsqrt(deg)
    dinv_ref[...] = dinv
    xs_ref[...] = x_ref[...] * dinv[:N]


_prescale = pl.pallas_call(
    _prescale_body,
    out_shape=(
        jax.ShapeDtypeStruct((N, F), jnp.float32),
        jax.ShapeDtypeStruct((NPAD, 1), jnp.float32),
    ),
)


def _layer1_body(dinv_ref, u_ref, xs_ref, w1a_ref, b1a_ref, w2a_ref, b2a_ref,
                 ct_ref):
    dinv = dinv_ref[...]
    y = (u_ref[0] + u_ref[1] + xs_ref[...]) * dinv
    h1 = jnp.maximum(
        jnp.dot(y, w1a_ref[...], preferred_element_type=jnp.float32)
        + b1a_ref[...], 0.0) * dinv
    h2 = jnp.maximum(
        jnp.dot(y, w2a_ref[...], preferred_element_type=jnp.float32)
        + b2a_ref[...], 0.0) * dinv
    ct_ref[0] = h1[:, :F]
    ct_ref[1] = h1[:, F:]
    ct_ref[2] = h2[:, :F]
    ct_ref[3] = h2[:, F:]


_layer1 = pl.pallas_call(
    _layer1_body,
    grid=(NSTEP,),
    in_specs=[
        pl.BlockSpec((RB, 1), lambda i: (i, 0)),
        pl.BlockSpec((NC, RB, F), lambda i: (0, i, 0)),
        pl.BlockSpec((RB, F), lambda i: (i, 0)),
        pl.BlockSpec((F, H), lambda i: (0, 0)),
        pl.BlockSpec((1, H), lambda i: (0, 0)),
        pl.BlockSpec((F, H), lambda i: (0, 0)),
        pl.BlockSpec((1, H), lambda i: (0, 0)),
    ],
    out_specs=pl.BlockSpec((4, RB, F), lambda i: (0, i, 0)),
    out_shape=jax.ShapeDtypeStruct((4, N, F), jnp.float32),
)


def _heads_body(dinv_ref, v_ref, ct_ref, batch_ref, action_ref,
                w1b_ref, b1b_ref, fw1_ref, fb1_ref,
                w2b_ref, b2b_ref, fw2_ref, fb2_ref,
                q1_ref, q2_ref, g1_acc, g2_acc, cnt_acc):
    i = pl.program_id(0)
    dinv = dinv_ref[...]
    pm = (batch_ref[...] == lax.broadcasted_iota(jnp.int32, (1, NG), 1)
          ).astype(jnp.float32)

    def z_head(v_a, v_b, ct_a, ct_b, w_ref, b_ref):
        v = jnp.concatenate([v_a[0] + v_a[1], v_b[0] + v_b[1]], axis=1)
        cc = jnp.concatenate([ct_a, ct_b], axis=1)
        y2 = (v + cc) * dinv
        return jnp.maximum(
            jnp.dot(y2, w_ref[...], preferred_element_type=jnp.float32)
            + b_ref[...], 0.0)

    z1 = z_head(v_ref[0], v_ref[1], ct_ref[0], ct_ref[1], w1b_ref, b1b_ref)
    z2 = z_head(v_ref[2], v_ref[3], ct_ref[2], ct_ref[3], w2b_ref, b2b_ref)

    dn = (((0,), (0,)), ((), ()))
    g1 = lax.dot_general(pm, z1, dn, preferred_element_type=jnp.float32)
    g2 = lax.dot_general(pm, z2, dn, preferred_element_type=jnp.float32)
    cnt = lax.dot_general(pm, jnp.ones((RB, 1), jnp.float32), dn,
                          preferred_element_type=jnp.float32)

    @pl.when(i == 0)
    def _init():
        g1_acc[...] = jnp.zeros_like(g1_acc)
        g2_acc[...] = jnp.zeros_like(g2_acc)
        cnt_acc[...] = jnp.zeros_like(cnt_acc)

    g1_acc[...] += g1
    g2_acc[...] += g2
    cnt_acc[...] += cnt

    @pl.when(i == NSTEP - 1)
    def _fin():
        inv_cnt = 1.0 / jnp.maximum(cnt_acc[...], 1.0)
        g1m = g1_acc[...] * inv_cnt
        g2m = g2_acc[...] * inv_cnt
        act = action_ref[...]
        q1_ref[...] = (jnp.dot(g1m, fw1_ref[:H, :],
                               preferred_element_type=jnp.float32)
                       + jnp.dot(act, fw1_ref[H:, :],
                                 preferred_element_type=jnp.float32)
                       + fb1_ref[...])
        q2_ref[...] = (jnp.dot(g2m, fw2_ref[:H, :],
                               preferred_element_type=jnp.float32)
                       + jnp.dot(act, fw2_ref[H:, :],
                                 preferred_element_type=jnp.float32)
                       + fb2_ref[...])


_heads = pl.pallas_call(
    _heads_body,
    grid=(NSTEP,),
    in_specs=[
        pl.BlockSpec((RB, 1), lambda i: (i, 0)),
        pl.BlockSpec((4, NC, RB, F), lambda i: (0, 0, i, 0)),
        pl.BlockSpec((4, RB, F), lambda i: (0, i, 0)),
        pl.BlockSpec((RB, 1), lambda i: (i, 0)),
        pl.BlockSpec((NG, AD), lambda i: (0, 0)),
        pl.BlockSpec((H, H), lambda i: (0, 0)),
        pl.BlockSpec((1, H), lambda i: (0, 0)),
        pl.BlockSpec((H + AD, 1), lambda i: (0, 0)),
        pl.BlockSpec((1, 1), lambda i: (0, 0)),
        pl.BlockSpec((H, H), lambda i: (0, 0)),
        pl.BlockSpec((1, H), lambda i: (0, 0)),
        pl.BlockSpec((H + AD, 1), lambda i: (0, 0)),
        pl.BlockSpec((1, 1), lambda i: (0, 0)),
    ],
    out_specs=(
        pl.BlockSpec((NG, 1), lambda i: (0, 0)),
        pl.BlockSpec((NG, 1), lambda i: (0, 0)),
    ),
    out_shape=(
        jax.ShapeDtypeStruct((NG, 1), jnp.float32),
        jax.ShapeDtypeStruct((NG, 1), jnp.float32),
    ),
    scratch_shapes=[
        pltpu.VMEM((NG, H), jnp.float32),
        pltpu.VMEM((NG, H), jnp.float32),
        pltpu.VMEM((NG, 1), jnp.float32),
    ],
)


def kernel(x, edge_index, batch, action, W1a, b1a, W1b, b1b, fcW1, fcb1,
           W2a, b2a, W2b, b2b, fcW2, fcb2):
    pad_src = jnp.broadcast_to(
        (jnp.arange(PADW, dtype=jnp.int32) * 41) % N, (NW, PADW))
    pad_dst = jnp.broadcast_to(
        N + jnp.arange(PADW, dtype=jnp.int32) % (NPAD - N), (NW, PADW))
    src = jnp.concatenate(
        [edge_index[0].astype(jnp.int32).reshape(NW, E // NW), pad_src],
        axis=1).reshape(NW, NB, K)
    dst = jnp.concatenate(
        [edge_index[1].astype(jnp.int32).reshape(NW, E // NW), pad_dst],
        axis=1).reshape(NW, NB, K)
    batch_col = batch.astype(jnp.int32).reshape(N, 1)
    zeros = jnp.zeros((NPAD, F), jnp.float32)

    deg2 = _deg_kernel(dst)
    xs, dinv = _prescale(deg2.reshape(NC, NPAD, 1), x)
    u = _spmv1(xs.reshape(1, N, F), src, dst, zeros)
    ct = _layer1(dinv, u[0], xs, W1a, b1a.reshape(1, H),
                 W2a, b2a.reshape(1, H))
    v = _spmv4(ct, src, dst, zeros)
    q1, q2 = _heads(dinv, v, ct, batch_col, action,
                    W1b, b1b.reshape(1, H), fcW1, fcb1.reshape(1, 1),
                    W2b, b2b.reshape(1, H), fcW2, fcb2.reshape(1, 1))
    return (q1, q2)


# speedup vs baseline: 1.2790x; 1.2790x over previous
import functools

import jax
import jax.numpy as jnp
from jax import lax
from jax.experimental import pallas as pl
from jax.experimental.pallas import tpu as pltpu
from jax.experimental.pallas import tpu_sc as plsc

N = 10000
E = 320000
F = 128
H = 256
NG = 64
AD = 32

NC = 2
NS = 16
NW = NC * NS
K = 128
NB = 80
NBH = NB // 2
EPWP = NB * K
PADW = EPWP - E // NW
NPAD = 10240
ZDEG = NPAD // NS
RPS = NPAD // NS
RB = 1000
NSTEP = N // RB

_mesh = plsc.VectorSubcoreMesh(core_axis_name="c", subcore_axis_name="s")


@functools.partial(
    pl.kernel,
    out_type=jax.ShapeDtypeStruct((NC, NPAD), jnp.float32),
    mesh=_mesh,
    scratch_types=[
        pltpu.VMEM((NB, K), jnp.int32),
        pltpu.VMEM((128,), jnp.float32),
        pltpu.VMEM((ZDEG,), jnp.float32),
        pltpu.VMEM_SHARED((NPAD,), jnp.float32),
    ],
)
def _deg_kernel(dst_hbm, out_hbm, idx_v, ones_v, zb_v, acc_sh):
    c = lax.axis_index("c")
    s = lax.axis_index("s")
    w = c * NS + s
    for i in range(8):
        ones_v[pl.ds(i * 16, 16)] = jnp.ones((16,), jnp.float32)
    for i in range(ZDEG // 16):
        zb_v[pl.ds(i * 16, 16)] = jnp.zeros((16,), jnp.float32)
    pltpu.sync_copy(dst_hbm.at[w], idx_v)
    pltpu.sync_copy(zb_v, acc_sh.at[pl.ds(s * ZDEG, ZDEG)])
    plsc.subcore_barrier()

    def body(j, carry):
        pltpu.sync_copy(ones_v, acc_sh.at[idx_v.at[j]], add=True)
        return carry

    lax.fori_loop(0, NB, body, 0)
    plsc.subcore_barrier()
    pltpu.sync_copy(acc_sh.at[pl.ds(s * ZDEG, ZDEG)],
                    out_hbm.at[c, pl.ds(s * ZDEG, ZDEG)])


def _make_spmv(ch_count):
    @functools.partial(
        pl.kernel,
        out_type=jax.ShapeDtypeStruct((ch_count, NC, NPAD, F), jnp.float32),
        mesh=_mesh,
        scratch_types=[
            pltpu.VMEM((NBH, K), jnp.int32),
            pltpu.VMEM((NBH, K), jnp.int32),
            pltpu.VMEM((2, K, F), jnp.float32),
            pltpu.VMEM_SHARED((NPAD, F), jnp.float32),
            pltpu.SemaphoreType.DMA,
            pltpu.SemaphoreType.DMA,
        ],
    )
    def _spmv(table_hbm, src_hbm, dst_hbm, zeros_hbm, out_hbm,
              src_v, dst_v, rows_v, acc_sh, sem0, sem1):
        c = lax.axis_index("c")
        s = lax.axis_index("s")
        w = c * NS + s
        sems = (sem0, sem1)

        for ch in range(ch_count):
            tab = table_hbm.at[ch]

            def gather(j, b):
                return pltpu.make_async_copy(tab.at[src_v.at[j]],
                                             rows_v.at[b], sems[b])

            pltpu.sync_copy(zeros_hbm.at[pl.ds(s * RPS, RPS)],
                            acc_sh.at[pl.ds(s * RPS, RPS)])
            plsc.subcore_barrier()

            for half in range(2):
                pltpu.sync_copy(src_hbm.at[w, pl.ds(half * NBH, NBH)], src_v)
                pltpu.sync_copy(dst_hbm.at[w, pl.ds(half * NBH, NBH)], dst_v)
                gather(0, 0).start()
                gather(1, 1).start()

                def body(i, carry):
                    j0 = 2 * i
                    gather(j0, 0).wait()
                    pltpu.sync_copy(rows_v.at[0], acc_sh.at[dst_v.at[j0]],
                                    add=True)

                    @pl.when(j0 + 2 < NBH)
                    def _():
                        gather(j0 + 2, 0).start()

                    gather(j0 + 1, 1).wait()
                    pltpu.sync_copy(rows_v.at[1], acc_sh.at[dst_v.at[j0 + 1]],
                                    add=True)

                    @pl.when(j0 + 3 < NBH)
                    def _():
                        gather(j0 + 3, 1).start()

                    return carry

                lax.fori_loop(0, NBH // 2, body, 0)
            plsc.subcore_barrier()
            pltpu.sync_copy(acc_sh.at[pl.ds(s * RPS, RPS)],
                            out_hbm.at[ch, c, pl.ds(s * RPS, RPS)])

    return _spmv


_spmv1 = _make_spmv(1)
_spmv4 = _make_spmv(4)


def _prescale_body(deg_ref, x_ref, xs_ref, dinv_ref):
    deg = deg_ref[0] + deg_ref[1] + 1.0
    dinv = lax.rsqrt(deg)
    dinv_ref[...] = dinv
    xs_ref[...] = x_ref[...] * dinv[:N]


_prescale = pl.pallas_call(
    _prescale_body,
    out_shape=(
        jax.ShapeDtypeStruct((N, F), jnp.float32),
        jax.ShapeDtypeStruct((NPAD, 1), jnp.float32),
    ),
)


def _layer1_body(dinv_ref, u_ref, xs_ref, w1a_ref, b1a_ref, w2a_ref, b2a_ref,
                 ct_ref):
    dinv = dinv_ref[...]
    y = (u_ref[0] + u_ref[1] + xs_ref[...]) * dinv
    h1 = jnp.maximum(
        jnp.dot(y, w1a_ref[...], preferred_element_type=jnp.float32)
        + b1a_ref[...], 0.0) * dinv
    h2 = jnp.maximum(
        jnp.dot(y, w2a_ref[...], preferred_element_type=jnp.float32)
        + b2a_ref[...], 0.0) * dinv
    ct_ref[0] = h1[:, :F]
    ct_ref[1] = h1[:, F:]
    ct_ref[2] = h2[:, :F]
    ct_ref[3] = h2[:, F:]


_layer1 = pl.pallas_call(
    _layer1_body,
    grid=(NSTEP,),
    in_specs=[
        pl.BlockSpec((RB, 1), lambda i: (i, 0)),
        pl.BlockSpec((NC, RB, F), lambda i: (0, i, 0)),
        pl.BlockSpec((RB, F), lambda i: (i, 0)),
        pl.BlockSpec((F, H), lambda i: (0, 0)),
        pl.BlockSpec((1, H), lambda i: (0, 0)),
        pl.BlockSpec((F, H), lambda i: (0, 0)),
        pl.BlockSpec((1, H), lambda i: (0, 0)),
    ],
    out_specs=pl.BlockSpec((4, RB, F), lambda i: (0, i, 0)),
    out_shape=jax.ShapeDtypeStruct((4, N, F), jnp.float32),
)


def _heads_body(dinv_ref, v_ref, ct_ref, batch_ref, action_ref,
                w1b_ref, b1b_ref, fw1_ref, fb1_ref,
                w2b_ref, b2b_ref, fw2_ref, fb2_ref,
                q1_ref, q2_ref, g1_acc, g2_acc, cnt_acc):
    i = pl.program_id(0)
    dinv = dinv_ref[...]
    pm = (batch_ref[...] == lax.broadcasted_iota(jnp.int32, (1, NG), 1)
          ).astype(jnp.float32)

    def z_head(v_a, v_b, ct_a, ct_b, w_ref, b_ref):
        v = jnp.concatenate([v_a[0] + v_a[1], v_b[0] + v_b[1]], axis=1)
        cc = jnp.concatenate([ct_a, ct_b], axis=1)
        y2 = (v + cc) * dinv
        return jnp.maximum(
            jnp.dot(y2, w_ref[...], preferred_element_type=jnp.float32)
            + b_ref[...], 0.0)

    z1 = z_head(v_ref[0], v_ref[1], ct_ref[0], ct_ref[1], w1b_ref, b1b_ref)
    z2 = z_head(v_ref[2], v_ref[3], ct_ref[2], ct_ref[3], w2b_ref, b2b_ref)

    dn = (((0,), (0,)), ((), ()))
    g1 = lax.dot_general(pm, z1, dn, preferred_element_type=jnp.float32)
    g2 = lax.dot_general(pm, z2, dn, preferred_element_type=jnp.float32)
    cnt = lax.dot_general(pm, jnp.ones((RB, 1), jnp.float32), dn,
                          preferred_element_type=jnp.float32)

    @pl.when(i == 0)
    def _init():
        g1_acc[...] = jnp.zeros_like(g1_acc)
        g2_acc[...] = jnp.zeros_like(g2_acc)
        cnt_acc[...] = jnp.zeros_like(cnt_acc)

    g1_acc[...] += g1
    g2_acc[...] += g2
    cnt_acc[...] += cnt

    @pl.when(i == NSTEP - 1)
    def _fin():
        inv_cnt = 1.0 / jnp.maximum(cnt_acc[...], 1.0)
        g1m = g1_acc[...] * inv_cnt
        g2m = g2_acc[...] * inv_cnt
        act = action_ref[...]
        q1_ref[...] = (jnp.dot(g1m, fw1_ref[:H, :],
                               preferred_element_type=jnp.float32)
                       + jnp.dot(act, fw1_ref[H:, :],
                                 preferred_element_type=jnp.float32)
                       + fb1_ref[...])
        q2_ref[...] = (jnp.dot(g2m, fw2_ref[:H, :],
                               preferred_element_type=jnp.float32)
                       + jnp.dot(act, fw2_ref[H:, :],
                                 preferred_element_type=jnp.float32)
                       + fb2_ref[...])


_heads = pl.pallas_call(
    _heads_body,
    grid=(NSTEP,),
    in_specs=[
        pl.BlockSpec((RB, 1), lambda i: (i, 0)),
        pl.BlockSpec((4, NC, RB, F), lambda i: (0, 0, i, 0)),
        pl.BlockSpec((4, RB, F), lambda i: (0, i, 0)),
        pl.BlockSpec((RB, 1), lambda i: (i, 0)),
        pl.BlockSpec((NG, AD), lambda i: (0, 0)),
        pl.BlockSpec((H, H), lambda i: (0, 0)),
        pl.BlockSpec((1, H), lambda i: (0, 0)),
        pl.BlockSpec((H + AD, 1), lambda i: (0, 0)),
        pl.BlockSpec((1, 1), lambda i: (0, 0)),
        pl.BlockSpec((H, H), lambda i: (0, 0)),
        pl.BlockSpec((1, H), lambda i: (0, 0)),
        pl.BlockSpec((H + AD, 1), lambda i: (0, 0)),
        pl.BlockSpec((1, 1), lambda i: (0, 0)),
    ],
    out_specs=(
        pl.BlockSpec((NG, 1), lambda i: (0, 0)),
        pl.BlockSpec((NG, 1), lambda i: (0, 0)),
    ),
    out_shape=(
        jax.ShapeDtypeStruct((NG, 1), jnp.float32),
        jax.ShapeDtypeStruct((NG, 1), jnp.float32),
    ),
    scratch_shapes=[
        pltpu.VMEM((NG, H), jnp.float32),
        pltpu.VMEM((NG, H), jnp.float32),
        pltpu.VMEM((NG, 1), jnp.float32),
    ],
)


def kernel(x, edge_index, batch, action, W1a, b1a, W1b, b1b, fcW1, fcb1,
           W2a, b2a, W2b, b2b, fcW2, fcb2):
    pad_src = jnp.broadcast_to(
        (jnp.arange(PADW, dtype=jnp.int32) * 41) % N, (NW, PADW))
    pad_dst = jnp.broadcast_to(
        N + jnp.arange(PADW, dtype=jnp.int32) % (NPAD - N), (NW, PADW))
    src = jnp.concatenate(
        [edge_index[0].astype(jnp.int32).reshape(NW, E // NW), pad_src],
        axis=1).reshape(NW, NB, K)
    dst = jnp.concatenate(
        [edge_index[1].astype(jnp.int32).reshape(NW, E // NW), pad_dst],
        axis=1).reshape(NW, NB, K)
    batch_col = batch.astype(jnp.int32).reshape(N, 1)
    zeros = jnp.zeros((NPAD, F), jnp.float32)

    deg2 = _deg_kernel(dst)
    xs, dinv = _prescale(deg2.reshape(NC, NPAD, 1), x)
    u = _spmv1(xs.reshape(1, N, F), src, dst, zeros)
    ct = _layer1(dinv, u[0], xs, W1a, b1a.reshape(1, H),
                 W2a, b2a.reshape(1, H))
    v = _spmv4(ct, src, dst, zeros)
    q1, q2 = _heads(dinv, v, ct, batch_col, action,
                    W1b, b1b.reshape(1, H), fcW1, fcb1.reshape(1, 1),
                    W2b, b2b.reshape(1, H), fcW2, fcb2.reshape(1, 1))
    return (q1, q2)
